# baseline (speedup 1.0000x reference)
import functools

import jax
import jax.numpy as jnp
from jax import lax
from jax.experimental import pallas as pl
from jax.experimental.pallas import tpu as pltpu
from jax.experimental.pallas import tpu_sc as plsc

N_NODES = 10000
D = 128
N_GRAPHS = 64

NC = 2
NS = 16
NW = NC * NS
CHUNK = 128
ROWS_PER_TILE = 640
N_PAD = NS * ROWS_PER_TILE
BLK = 1024
CORE0_FRACTION = 0.5


def _sc_aggregate(px, src2, dst2, k0, k1):
  mesh = plsc.VectorSubcoreMesh(core_axis_name="c", subcore_axis_name="s")
  kmax = max(k0, k1)
  t0 = NS * k0

  @functools.partial(
      pl.kernel,
      out_type=jax.ShapeDtypeStruct((NC, N_PAD, D), jnp.float32),
      mesh=mesh,
      scratch_types=[
          pltpu.MemorySpace.VMEM_SHARED((N_PAD, D), jnp.float32),
          pltpu.MemorySpace.VMEM((kmax, CHUNK), jnp.int32),
          pltpu.MemorySpace.VMEM((kmax, CHUNK), jnp.int32),
          pltpu.MemorySpace.VMEM((CHUNK, D), jnp.float32),
          pltpu.SemaphoreType.DMA,
      ],
  )
  def agg(px_hbm, src_hbm, dst_hbm, out_hbm,
          acc_sh, src_v, dst_v, rows_v, sem):
    c = lax.axis_index("c")
    s = lax.axis_index("s")
    r0 = s * ROWS_PER_TILE

    zv = jnp.zeros((16,), jnp.float32)

    def zstep(t, carry):
      rows_v[t >> 3, pl.ds((t & 7) * 16, 16)] = zv
      return carry

    lax.fori_loop(0, CHUNK * 8, zstep, 0)
    for q in range(ROWS_PER_TILE // CHUNK):
      pltpu.sync_copy(rows_v, acc_sh.at[pl.ds(r0 + q * CHUNK, CHUNK)])

    @pl.when(c == 0)
    def _():
      pltpu.sync_copy(src_hbm.at[pl.ds(s * k0, k0)], src_v.at[pl.ds(0, k0)])
      pltpu.sync_copy(dst_hbm.at[pl.ds(s * k0, k0)], dst_v.at[pl.ds(0, k0)])

    @pl.when(c == 1)
    def _():
      pltpu.sync_copy(src_hbm.at[pl.ds(t0 + s * k1, k1)],
                      src_v.at[pl.ds(0, k1)])
      pltpu.sync_copy(dst_hbm.at[pl.ds(t0 + s * k1, k1)],
                      dst_v.at[pl.ds(0, k1)])

    plsc.subcore_barrier()

    def step(j, carry):
      pltpu.async_copy(px_hbm.at[src_v.at[j]], rows_v, sem).wait()
      pltpu.sync_copy(rows_v, acc_sh.at[dst_v.at[j]], add=True)
      return carry

    kc = jnp.where(c == 0, k0, k1)
    lax.fori_loop(0, kc, step, 0)
    plsc.subcore_barrier()

    pltpu.sync_copy(acc_sh.at[pl.ds(r0, ROWS_PER_TILE)],
                    out_hbm.at[c].at[pl.ds(r0, ROWS_PER_TILE)])

  return agg(px, src2, dst2)


def _mlp_body(px_ref, a0_ref, a1_ref, wa_ref, ba_ref, wb_ref, bb_ref,
              out_ref):
  h = px_ref[...] + a0_ref[...] + a1_ref[...]
  h = jnp.maximum(
      jnp.dot(h, wa_ref[...], preferred_element_type=jnp.float32)
      + ba_ref[...], 0.0)
  h = jnp.maximum(
      jnp.dot(h, wb_ref[...], preferred_element_type=jnp.float32)
      + bb_ref[...], 0.0)
  out_ref[...] = h


def _mlp(px, a0, a1, Wa, ba, Wb, bb):
  n_blocks = N_PAD // BLK
  return pl.pallas_call(
      _mlp_body,
      grid=(n_blocks,),
      in_specs=[
          pl.BlockSpec((BLK, D), lambda i: (i, 0)),
          pl.BlockSpec((BLK, D), lambda i: (i, 0)),
          pl.BlockSpec((BLK, D), lambda i: (i, 0)),
          pl.BlockSpec((D, D), lambda i: (0, 0)),
          pl.BlockSpec((1, D), lambda i: (0, 0)),
          pl.BlockSpec((D, D), lambda i: (0, 0)),
          pl.BlockSpec((1, D), lambda i: (0, 0)),
      ],
      out_specs=pl.BlockSpec((BLK, D), lambda i: (i, 0)),
      out_shape=jax.ShapeDtypeStruct((N_PAD, D), jnp.float32),
  )(px, a0, a1, Wa, ba.reshape(1, D), Wb, bb.reshape(1, D))


def _mlp_pool_body(px_ref, a0_ref, a1_ref, wa_ref, ba_ref, wb_ref, bb_ref,
                   b_ref, pool_ref):
  h = px_ref[...] + a0_ref[...] + a1_ref[...]
  h = jnp.maximum(
      jnp.dot(h, wa_ref[...], preferred_element_type=jnp.float32)
      + ba_ref[...], 0.0)
  h = jnp.maximum(
      jnp.dot(h, wb_ref[...], preferred_element_type=jnp.float32)
      + bb_ref[...], 0.0)
  seg = b_ref[0, 0, :]
  onehot = (lax.broadcasted_iota(jnp.int32, (N_GRAPHS, BLK), 0)
            == seg[None, :]).astype(jnp.float32)

  @pl.when(pl.program_id(0) == 0)
  def _():
    pool_ref[...] = jnp.zeros_like(pool_ref)

  pool_ref[...] += jnp.dot(onehot, h, preferred_element_type=jnp.float32)


def _mlp_pool(px, a0, a1, Wa, ba, Wb, bb, batch3):
  n_blocks = N_PAD // BLK
  return pl.pallas_call(
      _mlp_pool_body,
      grid=(n_blocks,),
      in_specs=[
          pl.BlockSpec((BLK, D), lambda i: (i, 0)),
          pl.BlockSpec((BLK, D), lambda i: (i, 0)),
          pl.BlockSpec((BLK, D), lambda i: (i, 0)),
          pl.BlockSpec((D, D), lambda i: (0, 0)),
          pl.BlockSpec((1, D), lambda i: (0, 0)),
          pl.BlockSpec((D, D), lambda i: (0, 0)),
          pl.BlockSpec((1, D), lambda i: (0, 0)),
          pl.BlockSpec((1, 1, BLK), lambda i: (i, 0, 0)),
      ],
      out_specs=pl.BlockSpec((N_GRAPHS, D), lambda i: (0, 0)),
      out_shape=jax.ShapeDtypeStruct((N_GRAPHS, D), jnp.float32),
  )(px, a0, a1, Wa, ba.reshape(1, D), Wb, bb.reshape(1, D), batch3)


def _head_body(p_ref, w1_ref, b1_ref, w2_ref, b2_ref, out_ref):
  h = jnp.maximum(
      jnp.dot(p_ref[...], w1_ref[...], preferred_element_type=jnp.float32)
      + b1_ref[...], 0.0)
  out_ref[...] = (
      jnp.dot(h, w2_ref[...], preferred_element_type=jnp.float32)
      + b2_ref[...])


def _head(pooled, Wl1, bl1, Wl2p, bl2b):
  return pl.pallas_call(
      _head_body,
      in_specs=[
          pl.BlockSpec((N_GRAPHS, D), lambda: (0, 0)),
          pl.BlockSpec((D, D), lambda: (0, 0)),
          pl.BlockSpec((1, D), lambda: (0, 0)),
          pl.BlockSpec((D, D), lambda: (0, 0)),
          pl.BlockSpec((1, D), lambda: (0, 0)),
      ],
      out_specs=pl.BlockSpec((N_GRAPHS, D), lambda: (0, 0)),
      out_shape=jax.ShapeDtypeStruct((N_GRAPHS, D), jnp.float32),
  )(pooled, Wl1, bl1.reshape(1, D), Wl2p, bl2b)


def kernel(x, edge_index, batch, W1a, b1a, W1b, b1b, W2a, b2a, W2b, b2b,
           Wl1, bl1, Wl2, bl2):
  n_edges = edge_index.shape[1]
  t_chunks = -(-n_edges // CHUNK)
  k0 = max(8 * round(t_chunks * CORE0_FRACTION / NS / 8), 0)
  k1 = -(-(t_chunks - NS * k0) // (NS * 8)) * 8
  tp = NS * (k0 + k1)
  ep = tp * CHUNK

  src = edge_index[0].astype(jnp.int32)
  dst = edge_index[1].astype(jnp.int32)
  pad = ep - n_edges
  src_p = jnp.concatenate([src, jnp.zeros((pad,), jnp.int32)])
  dump = N_NODES + jnp.arange(pad, dtype=jnp.int32) % (N_PAD - N_NODES)
  dst_p = jnp.concatenate([dst, dump])
  src2 = src_p.reshape(tp, CHUNK)
  dst2 = dst_p.reshape(tp, CHUNK)

  px = jnp.concatenate(
      [x, jnp.zeros((N_PAD - N_NODES, D), jnp.float32)], axis=0)

  batch_p = jnp.concatenate([
      batch.astype(jnp.int32),
      jnp.full((N_PAD - N_NODES,), N_GRAPHS, jnp.int32)
  ])
  batch3 = batch_p.reshape(N_PAD // BLK, 1, BLK)

  acc1 = _sc_aggregate(px, src2, dst2, k0, k1)
  h1 = _mlp(px, acc1[0], acc1[1], W1a, b1a, W1b, b1b)
  acc2 = _sc_aggregate(h1, src2, dst2, k0, k1)
  pooled = _mlp_pool(h1, acc2[0], acc2[1], W2a, b2a, W2b, b2b, batch3)

  Wl2p = jnp.pad(Wl2, ((0, 0), (0, D - Wl2.shape[1])))
  bl2b = jnp.broadcast_to(bl2.reshape(1, 1), (1, D))
  out = _head(pooled, Wl1, bl1, Wl2p, bl2b)
  return out[:, :1]


# speedup vs baseline: 3.6820x; 3.6820x over previous
import functools

import jax
import jax.numpy as jnp
from jax import lax
from jax.experimental import pallas as pl
from jax.experimental.pallas import tpu as pltpu
from jax.experimental.pallas import tpu_sc as plsc

N_NODES = 10000
D = 128
N_GRAPHS = 64

NC = 2
NS = 16
NW = NC * NS
CHUNK = 128
ROWS_PER_TILE = 640
N_PAD = NS * ROWS_PER_TILE
BLK = 1024
CORE0_FRACTION = 0.5
IH = 40


def _sc_aggregate(px, src2, dst2, k0, k1):
  mesh = plsc.VectorSubcoreMesh(core_axis_name="c", subcore_axis_name="s")
  kmax = max(k0, k1)
  t0 = NS * k0

  assert k0 == k1 and k0 % IH == 0 and IH % 2 == 0
  nh = k0 // IH

  @functools.partial(
      pl.kernel,
      out_type=jax.ShapeDtypeStruct((NC, N_PAD, D), jnp.float32),
      mesh=mesh,
      scratch_types=[
          pltpu.MemorySpace.VMEM_SHARED((N_PAD, D), jnp.float32),
          pltpu.MemorySpace.VMEM((IH, CHUNK), jnp.int32),
          pltpu.MemorySpace.VMEM((IH, CHUNK), jnp.int32),
          pltpu.MemorySpace.VMEM((CHUNK, D), jnp.float32),
          pltpu.MemorySpace.VMEM((CHUNK, D), jnp.float32),
          pltpu.SemaphoreType.DMA,
          pltpu.SemaphoreType.DMA,
      ],
  )
  def agg(px_hbm, src_hbm, dst_hbm, out_hbm,
          acc_sh, src_v, dst_v, rows0, rows1, gsem0, gsem1):
    c = lax.axis_index("c")
    s = lax.axis_index("s")
    r0 = s * ROWS_PER_TILE
    base = c * t0 + s * k0

    zv = jnp.zeros((16,), jnp.float32)

    def zstep(t, carry):
      rows0[t >> 3, pl.ds((t & 7) * 16, 16)] = zv
      return carry

    lax.fori_loop(0, CHUNK * 8, zstep, 0)
    for q in range(ROWS_PER_TILE // CHUNK):
      pltpu.sync_copy(rows0, acc_sh.at[pl.ds(r0 + q * CHUNK, CHUNK)])
    plsc.subcore_barrier()

    for h in range(nh):
      pltpu.sync_copy(src_hbm.at[pl.ds(base + h * IH, IH)], src_v)
      pltpu.sync_copy(dst_hbm.at[pl.ds(base + h * IH, IH)], dst_v)
      pltpu.async_copy(px_hbm.at[src_v.at[0]], rows0, gsem0)

      def step(half, carry):
        jj = half * 2
        d1 = pltpu.async_copy(px_hbm.at[src_v.at[jj + 1]], rows1, gsem1)
        pltpu.make_async_copy(px_hbm.at[pl.ds(0, CHUNK)], rows0,
                              gsem0).wait()
        pltpu.sync_copy(rows0, acc_sh.at[dst_v.at[jj]], add=True)

        @pl.when(jj + 2 < IH)
        def _():
          pltpu.async_copy(px_hbm.at[src_v.at[jj + 2]], rows0, gsem0)

        d1.wait()
        pltpu.sync_copy(rows1, acc_sh.at[dst_v.at[jj + 1]], add=True)
        return carry

      lax.fori_loop(0, IH // 2, step, 0)
    plsc.subcore_barrier()

    pltpu.sync_copy(acc_sh.at[pl.ds(r0, ROWS_PER_TILE)],
                    out_hbm.at[c].at[pl.ds(r0, ROWS_PER_TILE)])

  return agg(px, src2, dst2)


def _mlp_body(px_ref, a0_ref, a1_ref, wa_ref, ba_ref, wb_ref, bb_ref,
              out_ref):
  h = px_ref[...] + a0_ref[...] + a1_ref[...]
  h = jnp.maximum(
      jnp.dot(h, wa_ref[...], preferred_element_type=jnp.float32)
      + ba_ref[...], 0.0)
  h = jnp.maximum(
      jnp.dot(h, wb_ref[...], preferred_element_type=jnp.float32)
      + bb_ref[...], 0.0)
  out_ref[...] = h


def _mlp(px, a0, a1, Wa, ba, Wb, bb):
  n_blocks = N_PAD // BLK
  return pl.pallas_call(
      _mlp_body,
      grid=(n_blocks,),
      in_specs=[
          pl.BlockSpec((BLK, D), lambda i: (i, 0)),
          pl.BlockSpec((BLK, D), lambda i: (i, 0)),
          pl.BlockSpec((BLK, D), lambda i: (i, 0)),
          pl.BlockSpec((D, D), lambda i: (0, 0)),
          pl.BlockSpec((1, D), lambda i: (0, 0)),
          pl.BlockSpec((D, D), lambda i: (0, 0)),
          pl.BlockSpec((1, D), lambda i: (0, 0)),
      ],
      out_specs=pl.BlockSpec((BLK, D), lambda i: (i, 0)),
      out_shape=jax.ShapeDtypeStruct((N_PAD, D), jnp.float32),
  )(px, a0, a1, Wa, ba.reshape(1, D), Wb, bb.reshape(1, D))


def _mlp_pool_body(px_ref, a0_ref, a1_ref, wa_ref, ba_ref, wb_ref, bb_ref,
                   b_ref, pool_ref):
  h = px_ref[...] + a0_ref[...] + a1_ref[...]
  h = jnp.maximum(
      jnp.dot(h, wa_ref[...], preferred_element_type=jnp.float32)
      + ba_ref[...], 0.0)
  h = jnp.maximum(
      jnp.dot(h, wb_ref[...], preferred_element_type=jnp.float32)
      + bb_ref[...], 0.0)
  seg = b_ref[0, 0, :]
  onehot = (lax.broadcasted_iota(jnp.int32, (N_GRAPHS, BLK), 0)
            == seg[None, :]).astype(jnp.float32)

  @pl.when(pl.program_id(0) == 0)
  def _():
    pool_ref[...] = jnp.zeros_like(pool_ref)

  pool_ref[...] += jnp.dot(onehot, h, preferred_element_type=jnp.float32)


def _mlp_pool(px, a0, a1, Wa, ba, Wb, bb, batch3):
  n_blocks = N_PAD // BLK
  return pl.pallas_call(
      _mlp_pool_body,
      grid=(n_blocks,),
      in_specs=[
          pl.BlockSpec((BLK, D), lambda i: (i, 0)),
          pl.BlockSpec((BLK, D), lambda i: (i, 0)),
          pl.BlockSpec((BLK, D), lambda i: (i, 0)),
          pl.BlockSpec((D, D), lambda i: (0, 0)),
          pl.BlockSpec((1, D), lambda i: (0, 0)),
          pl.BlockSpec((D, D), lambda i: (0, 0)),
          pl.BlockSpec((1, D), lambda i: (0, 0)),
          pl.BlockSpec((1, 1, BLK), lambda i: (i, 0, 0)),
      ],
      out_specs=pl.BlockSpec((N_GRAPHS, D), lambda i: (0, 0)),
      out_shape=jax.ShapeDtypeStruct((N_GRAPHS, D), jnp.float32),
  )(px, a0, a1, Wa, ba.reshape(1, D), Wb, bb.reshape(1, D), batch3)


def _head_body(p_ref, w1_ref, b1_ref, w2_ref, b2_ref, out_ref):
  h = jnp.maximum(
      jnp.dot(p_ref[...], w1_ref[...], preferred_element_type=jnp.float32)
      + b1_ref[...], 0.0)
  out_ref[...] = (
      jnp.dot(h, w2_ref[...], preferred_element_type=jnp.float32)
      + b2_ref[...])


def _head(pooled, Wl1, bl1, Wl2p, bl2b):
  return pl.pallas_call(
      _head_body,
      in_specs=[
          pl.BlockSpec((N_GRAPHS, D), lambda: (0, 0)),
          pl.BlockSpec((D, D), lambda: (0, 0)),
          pl.BlockSpec((1, D), lambda: (0, 0)),
          pl.BlockSpec((D, D), lambda: (0, 0)),
          pl.BlockSpec((1, D), lambda: (0, 0)),
      ],
      out_specs=pl.BlockSpec((N_GRAPHS, D), lambda: (0, 0)),
      out_shape=jax.ShapeDtypeStruct((N_GRAPHS, D), jnp.float32),
  )(pooled, Wl1, bl1.reshape(1, D), Wl2p, bl2b)


def kernel(x, edge_index, batch, W1a, b1a, W1b, b1b, W2a, b2a, W2b, b2b,
           Wl1, bl1, Wl2, bl2):
  n_edges = edge_index.shape[1]
  t_chunks = -(-n_edges // CHUNK)
  k0 = max(8 * round(t_chunks * CORE0_FRACTION / NS / 8), 0)
  k1 = -(-(t_chunks - NS * k0) // (NS * 8)) * 8
  tp = NS * (k0 + k1)
  ep = tp * CHUNK

  src = edge_index[0].astype(jnp.int32)
  dst = edge_index[1].astype(jnp.int32)
  pad = ep - n_edges
  ar = jnp.arange(pad, dtype=jnp.int32)
  src_p = jnp.concatenate([src, ar % N_NODES])
  dump = N_NODES + ar % (N_PAD - N_NODES)
  dst_p = jnp.concatenate([dst, dump])
  src2 = src_p.reshape(tp, CHUNK)
  dst2 = dst_p.reshape(tp, CHUNK)

  px = jnp.concatenate(
      [x, jnp.zeros((N_PAD - N_NODES, D), jnp.float32)], axis=0)

  batch_p = jnp.concatenate([
      batch.astype(jnp.int32),
      jnp.full((N_PAD - N_NODES,), N_GRAPHS, jnp.int32)
  ])
  batch3 = batch_p.reshape(N_PAD // BLK, 1, BLK)

  acc1 = _sc_aggregate(px, src2, dst2, k0, k1)
  h1 = _mlp(px, acc1[0], acc1[1], W1a, b1a, W1b, b1b)
  acc2 = _sc_aggregate(h1, src2, dst2, k0, k1)
  pooled = _mlp_pool(h1, acc2[0], acc2[1], W2a, b2a, W2b, b2b, batch3)

  Wl2p = jnp.pad(Wl2, ((0, 0), (0, D - Wl2.shape[1])))
  bl2b = jnp.broadcast_to(bl2.reshape(1, 1), (1, D))
  out = _head(pooled, Wl1, bl1, Wl2p, bl2b)
  return out[:, :1]


# speedup vs baseline: 3.7298x; 1.0130x over previous
import functools

import jax
import jax.numpy as jnp
from jax import lax
from jax.experimental import pallas as pl
from jax.experimental.pallas import tpu as pltpu
from jax.experimental.pallas import tpu_sc as plsc

N_NODES = 10000
D = 128
N_GRAPHS = 64

NC = 2
NS = 16
NW = NC * NS
CHUNK = 128
ROWS_PER_TILE = 640
N_PAD = NS * ROWS_PER_TILE
BLK = 1000
CORE0_FRACTION = 0.5
IH = 40


def _sc_aggregate(px, src2, dst2, k0, k1):
  mesh = plsc.VectorSubcoreMesh(core_axis_name="c", subcore_axis_name="s")
  kmax = max(k0, k1)
  t0 = NS * k0

  assert k0 == k1 and k0 % IH == 0 and IH % 2 == 0
  nh = k0 // IH

  @functools.partial(
      pl.kernel,
      out_type=jax.ShapeDtypeStruct((NC, N_NODES, D), jnp.float32),
      mesh=mesh,
      scratch_types=[
          pltpu.MemorySpace.VMEM_SHARED((N_PAD, D), jnp.float32),
          pltpu.MemorySpace.VMEM((IH, CHUNK), jnp.int32),
          pltpu.MemorySpace.VMEM((IH, CHUNK), jnp.int32),
          pltpu.MemorySpace.VMEM((CHUNK, D), jnp.float32),
          pltpu.MemorySpace.VMEM((CHUNK, D), jnp.float32),
          pltpu.SemaphoreType.DMA,
          pltpu.SemaphoreType.DMA,
      ],
  )
  def agg(px_hbm, src_hbm, dst_hbm, out_hbm,
          acc_sh, src_v, dst_v, rows0, rows1, gsem0, gsem1):
    c = lax.axis_index("c")
    s = lax.axis_index("s")
    r0 = s * ROWS_PER_TILE
    base = c * t0 + s * k0

    zv = jnp.zeros((16,), jnp.float32)

    def zstep(t, carry):
      rows0[t >> 3, pl.ds((t & 7) * 16, 16)] = zv
      return carry

    lax.fori_loop(0, CHUNK * 8, zstep, 0)
    for q in range(ROWS_PER_TILE // CHUNK):
      pltpu.sync_copy(rows0, acc_sh.at[pl.ds(r0 + q * CHUNK, CHUNK)])
    plsc.subcore_barrier()

    for h in range(nh):
      pltpu.sync_copy(src_hbm.at[pl.ds(base + h * IH, IH)], src_v)
      pltpu.sync_copy(dst_hbm.at[pl.ds(base + h * IH, IH)], dst_v)
      pltpu.async_copy(px_hbm.at[src_v.at[0]], rows0, gsem0)

      def step(half, carry):
        jj = half * 2
        d1 = pltpu.async_copy(px_hbm.at[src_v.at[jj + 1]], rows1, gsem1)
        pltpu.make_async_copy(px_hbm.at[pl.ds(0, CHUNK)], rows0,
                              gsem0).wait()
        pltpu.sync_copy(rows0, acc_sh.at[dst_v.at[jj]], add=True)

        @pl.when(jj + 2 < IH)
        def _():
          pltpu.async_copy(px_hbm.at[src_v.at[jj + 2]], rows0, gsem0)

        d1.wait()
        pltpu.sync_copy(rows1, acc_sh.at[dst_v.at[jj + 1]], add=True)
        return carry

      lax.fori_loop(0, IH // 2, step, 0)
    plsc.subcore_barrier()

    @pl.when(s < NS - 1)
    def _():
      pltpu.sync_copy(acc_sh.at[pl.ds(r0, ROWS_PER_TILE)],
                      out_hbm.at[c].at[pl.ds(r0, ROWS_PER_TILE)])

    last = N_NODES - (NS - 1) * ROWS_PER_TILE

    @pl.when(s == NS - 1)
    def _():
      pltpu.sync_copy(acc_sh.at[pl.ds((NS - 1) * ROWS_PER_TILE, last)],
                      out_hbm.at[c].at[pl.ds((NS - 1) * ROWS_PER_TILE, last)])

  return agg(px, src2, dst2)


def _mlp_body(px_ref, a0_ref, a1_ref, wa_ref, ba_ref, wb_ref, bb_ref,
              out_ref):
  h = px_ref[...] + a0_ref[...] + a1_ref[...]
  h = jnp.maximum(
      jnp.dot(h, wa_ref[...], preferred_element_type=jnp.float32)
      + ba_ref[...], 0.0)
  h = jnp.maximum(
      jnp.dot(h, wb_ref[...], preferred_element_type=jnp.float32)
      + bb_ref[...], 0.0)
  out_ref[...] = h


def _mlp(px, a0, a1, Wa, ba, Wb, bb):
  n_blocks = N_NODES // BLK
  return pl.pallas_call(
      _mlp_body,
      grid=(n_blocks,),
      in_specs=[
          pl.BlockSpec((BLK, D), lambda i: (i, 0)),
          pl.BlockSpec((BLK, D), lambda i: (i, 0)),
          pl.BlockSpec((BLK, D), lambda i: (i, 0)),
          pl.BlockSpec((D, D), lambda i: (0, 0)),
          pl.BlockSpec((1, D), lambda i: (0, 0)),
          pl.BlockSpec((D, D), lambda i: (0, 0)),
          pl.BlockSpec((1, D), lambda i: (0, 0)),
      ],
      out_specs=pl.BlockSpec((BLK, D), lambda i: (i, 0)),
      out_shape=jax.ShapeDtypeStruct((N_NODES, D), jnp.float32),
  )(px, a0, a1, Wa, ba.reshape(1, D), Wb, bb.reshape(1, D))


def _mlp_pool_head_body(px_ref, a0_ref, a1_ref, wa_ref, ba_ref, wb_ref,
                        bb_ref, b_ref, w1_ref, b1_ref, w2_ref, b2_ref,
                        pool_ref, out_ref):
  h = px_ref[...] + a0_ref[...] + a1_ref[...]
  h = jnp.maximum(
      jnp.dot(h, wa_ref[...], preferred_element_type=jnp.float32)
      + ba_ref[...], 0.0)
  h = jnp.maximum(
      jnp.dot(h, wb_ref[...], preferred_element_type=jnp.float32)
      + bb_ref[...], 0.0)
  seg = b_ref[0, 0, :]
  onehot = (lax.broadcasted_iota(jnp.int32, (N_GRAPHS, BLK), 0)
            == seg[None, :]).astype(jnp.float32)

  @pl.when(pl.program_id(0) == 0)
  def _():
    pool_ref[...] = jnp.zeros_like(pool_ref)

  pool_ref[...] += jnp.dot(onehot, h, preferred_element_type=jnp.float32)

  @pl.when(pl.program_id(0) == pl.num_programs(0) - 1)
  def _():
    g = jnp.maximum(
        jnp.dot(pool_ref[...], w1_ref[...],
                preferred_element_type=jnp.float32) + b1_ref[...], 0.0)
    out_ref[...] = (
        jnp.dot(g, w2_ref[...], preferred_element_type=jnp.float32)
        + b2_ref[...])


def _mlp_pool_head(px, a0, a1, Wa, ba, Wb, bb, batch3, Wl1, bl1, Wl2p, bl2b):
  n_blocks = N_NODES // BLK
  full = lambda i: (0, 0)
  _, out = pl.pallas_call(
      _mlp_pool_head_body,
      grid=(n_blocks,),
      in_specs=[
          pl.BlockSpec((BLK, D), lambda i: (i, 0)),
          pl.BlockSpec((BLK, D), lambda i: (i, 0)),
          pl.BlockSpec((BLK, D), lambda i: (i, 0)),
          pl.BlockSpec((D, D), full),
          pl.BlockSpec((1, D), full),
          pl.BlockSpec((D, D), full),
          pl.BlockSpec((1, D), full),
          pl.BlockSpec((1, 1, BLK), lambda i: (i, 0, 0)),
          pl.BlockSpec((D, D), full),
          pl.BlockSpec((1, D), full),
          pl.BlockSpec((D, D), full),
          pl.BlockSpec((1, D), full),
      ],
      out_specs=[
          pl.BlockSpec((N_GRAPHS, D), full),
          pl.BlockSpec((N_GRAPHS, D), full),
      ],
      out_shape=[
          jax.ShapeDtypeStruct((N_GRAPHS, D), jnp.float32),
          jax.ShapeDtypeStruct((N_GRAPHS, D), jnp.float32),
      ],
  )(px, a0, a1, Wa, ba.reshape(1, D), Wb, bb.reshape(1, D), batch3,
    Wl1, bl1.reshape(1, D), Wl2p, bl2b)
  return out


def kernel(x, edge_index, batch, W1a, b1a, W1b, b1b, W2a, b2a, W2b, b2b,
           Wl1, bl1, Wl2, bl2):
  n_edges = edge_index.shape[1]
  t_chunks = -(-n_edges // CHUNK)
  k0 = max(8 * round(t_chunks * CORE0_FRACTION / NS / 8), 0)
  k1 = -(-(t_chunks - NS * k0) // (NS * 8)) * 8
  tp = NS * (k0 + k1)
  ep = tp * CHUNK

  src = edge_index[0].astype(jnp.int32)
  dst = edge_index[1].astype(jnp.int32)
  pad = ep - n_edges
  ar = jnp.arange(pad, dtype=jnp.int32)
  src_p = jnp.concatenate([src, ar % N_NODES])
  dump = N_NODES + ar % (N_PAD - N_NODES)
  dst_p = jnp.concatenate([dst, dump])
  src2 = src_p.reshape(tp, CHUNK)
  dst2 = dst_p.reshape(tp, CHUNK)

  batch3 = batch.astype(jnp.int32).reshape(N_NODES // BLK, 1, BLK)
  Wl2p = jnp.pad(Wl2, ((0, 0), (0, D - Wl2.shape[1])))
  bl2b = jnp.broadcast_to(bl2.reshape(1, 1), (1, D))

  acc1 = _sc_aggregate(x, src2, dst2, k0, k1)
  h1 = _mlp(x, acc1[0], acc1[1], W1a, b1a, W1b, b1b)
  acc2 = _sc_aggregate(h1, src2, dst2, k0, k1)
  out = _mlp_pool_head(h1, acc2[0], acc2[1], W2a, b2a, W2b, b2b, batch3,
                       Wl1, bl1, Wl2p, bl2b)
  return out[:, :1]


# speedup vs baseline: 3.7375x; 1.0021x over previous
import functools

import jax
import jax.numpy as jnp
from jax import lax
from jax.experimental import pallas as pl
from jax.experimental.pallas import tpu as pltpu
from jax.experimental.pallas import tpu_sc as plsc

N_NODES = 10000
D = 128
N_GRAPHS = 64

NC = 2
NS = 16
NW = NC * NS
CHUNK = 128
ROWS_PER_TILE = 640
N_PAD = NS * ROWS_PER_TILE
BLK = 1000
IH = 40


def _sc_aggregate(px, src2, dst2, kpt):
  mesh = plsc.VectorSubcoreMesh(core_axis_name="c", subcore_axis_name="s")
  t0 = NS * kpt

  assert kpt % IH == 0 and IH % 2 == 0
  nh = kpt // IH

  @functools.partial(
      pl.kernel,
      out_type=jax.ShapeDtypeStruct((NC, N_NODES, D), jnp.float32),
      mesh=mesh,
      scratch_types=[
          pltpu.MemorySpace.VMEM_SHARED((N_PAD, D), jnp.float32),
          pltpu.MemorySpace.VMEM((IH, CHUNK), jnp.int32),
          pltpu.MemorySpace.VMEM((IH, CHUNK), jnp.int32),
          pltpu.MemorySpace.VMEM((CHUNK, D), jnp.float32),
          pltpu.MemorySpace.VMEM((CHUNK, D), jnp.float32),
          pltpu.SemaphoreType.DMA,
          pltpu.SemaphoreType.DMA,
      ],
  )
  def agg(px_hbm, src_hbm, dst_hbm, out_hbm,
          acc_sh, src_v, dst_v, rows0, rows1, gsem0, gsem1):
    c = lax.axis_index("c")
    s = lax.axis_index("s")
    r0 = s * ROWS_PER_TILE
    base = c * t0 + s * kpt

    zv = jnp.zeros((16,), jnp.float32)

    def zstep(t, carry):
      rows0[t >> 3, pl.ds((t & 7) * 16, 16)] = zv
      return carry

    lax.fori_loop(0, CHUNK * 8, zstep, 0)
    for q in range(ROWS_PER_TILE // CHUNK):
      pltpu.sync_copy(rows0, acc_sh.at[pl.ds(r0 + q * CHUNK, CHUNK)])
    plsc.subcore_barrier()

    for h in range(nh):
      pltpu.sync_copy(src_hbm.at[pl.ds(base + h * IH, IH)], src_v)
      pltpu.sync_copy(dst_hbm.at[pl.ds(base + h * IH, IH)], dst_v)
      pltpu.async_copy(px_hbm.at[src_v.at[0]], rows0, gsem0)

      def step(half, carry):
        jj = half * 2
        d1 = pltpu.async_copy(px_hbm.at[src_v.at[jj + 1]], rows1, gsem1)
        pltpu.make_async_copy(px_hbm.at[pl.ds(0, CHUNK)], rows0,
                              gsem0).wait()
        pltpu.sync_copy(rows0, acc_sh.at[dst_v.at[jj]], add=True)

        @pl.when(jj + 2 < IH)
        def _():
          pltpu.async_copy(px_hbm.at[src_v.at[jj + 2]], rows0, gsem0)

        d1.wait()
        pltpu.sync_copy(rows1, acc_sh.at[dst_v.at[jj + 1]], add=True)
        return carry

      lax.fori_loop(0, IH // 2, step, 0)
    plsc.subcore_barrier()

    @pl.when(s < NS - 1)
    def _():
      pltpu.sync_copy(acc_sh.at[pl.ds(r0, ROWS_PER_TILE)],
                      out_hbm.at[c].at[pl.ds(r0, ROWS_PER_TILE)])

    last = N_NODES - (NS - 1) * ROWS_PER_TILE

    @pl.when(s == NS - 1)
    def _():
      pltpu.sync_copy(acc_sh.at[pl.ds((NS - 1) * ROWS_PER_TILE, last)],
                      out_hbm.at[c].at[pl.ds((NS - 1) * ROWS_PER_TILE, last)])

  return agg(px, src2, dst2)


def _mlp_body(px_ref, a0_ref, a1_ref, wa_ref, ba_ref, wb_ref, bb_ref,
              out_ref):
  h = px_ref[...] + a0_ref[...] + a1_ref[...]
  h = jnp.maximum(
      jnp.dot(h, wa_ref[...], preferred_element_type=jnp.float32)
      + ba_ref[...], 0.0)
  h = jnp.maximum(
      jnp.dot(h, wb_ref[...], preferred_element_type=jnp.float32)
      + bb_ref[...], 0.0)
  out_ref[...] = h


def _mlp(px, a0, a1, Wa, ba, Wb, bb):
  n_blocks = N_NODES // BLK
  return pl.pallas_call(
      _mlp_body,
      grid=(n_blocks,),
      in_specs=[
          pl.BlockSpec((BLK, D), lambda i: (i, 0)),
          pl.BlockSpec((BLK, D), lambda i: (i, 0)),
          pl.BlockSpec((BLK, D), lambda i: (i, 0)),
          pl.BlockSpec((D, D), lambda i: (0, 0)),
          pl.BlockSpec((1, D), lambda i: (0, 0)),
          pl.BlockSpec((D, D), lambda i: (0, 0)),
          pl.BlockSpec((1, D), lambda i: (0, 0)),
      ],
      out_specs=pl.BlockSpec((BLK, D), lambda i: (i, 0)),
      out_shape=jax.ShapeDtypeStruct((N_NODES, D), jnp.float32),
  )(px, a0, a1, Wa, ba.reshape(1, D), Wb, bb.reshape(1, D))


def _mlp_pool_head_body(px_ref, a0_ref, a1_ref, wa_ref, ba_ref, wb_ref,
                        bb_ref, b_ref, w1_ref, b1_ref, w2_ref, b2_ref,
                        pool_ref, out_ref):
  h = px_ref[...] + a0_ref[...] + a1_ref[...]
  h = jnp.maximum(
      jnp.dot(h, wa_ref[...], preferred_element_type=jnp.float32)
      + ba_ref[...], 0.0)
  h = jnp.maximum(
      jnp.dot(h, wb_ref[...], preferred_element_type=jnp.float32)
      + bb_ref[...], 0.0)
  seg = b_ref[0, 0, :]
  onehot = (lax.broadcasted_iota(jnp.int32, (N_GRAPHS, BLK), 0)
            == seg[None, :]).astype(jnp.float32)

  @pl.when(pl.program_id(0) == 0)
  def _():
    pool_ref[...] = jnp.zeros_like(pool_ref)

  pool_ref[...] += jnp.dot(onehot, h, preferred_element_type=jnp.float32)

  @pl.when(pl.program_id(0) == pl.num_programs(0) - 1)
  def _():
    g = jnp.maximum(
        jnp.dot(pool_ref[...], w1_ref[...],
                preferred_element_type=jnp.float32) + b1_ref[...], 0.0)
    out_ref[...] = (
        jnp.dot(g, w2_ref[...], preferred_element_type=jnp.float32)
        + b2_ref[...])


def _mlp_pool_head(px, a0, a1, Wa, ba, Wb, bb, batch3, Wl1, bl1, Wl2p, bl2b):
  n_blocks = N_NODES // BLK
  full = lambda i: (0, 0)
  _, out = pl.pallas_call(
      _mlp_pool_head_body,
      grid=(n_blocks,),
      in_specs=[
          pl.BlockSpec((BLK, D), lambda i: (i, 0)),
          pl.BlockSpec((BLK, D), lambda i: (i, 0)),
          pl.BlockSpec((BLK, D), lambda i: (i, 0)),
          pl.BlockSpec((D, D), full),
          pl.BlockSpec((1, D), full),
          pl.BlockSpec((D, D), full),
          pl.BlockSpec((1, D), full),
          pl.BlockSpec((1, 1, BLK), lambda i: (i, 0, 0)),
          pl.BlockSpec((D, D), full),
          pl.BlockSpec((1, D), full),
          pl.BlockSpec((D, D), full),
          pl.BlockSpec((1, D), full),
      ],
      out_specs=[
          pl.BlockSpec((N_GRAPHS, D), full),
          pl.BlockSpec((N_GRAPHS, D), full),
      ],
      out_shape=[
          jax.ShapeDtypeStruct((N_GRAPHS, D), jnp.float32),
          jax.ShapeDtypeStruct((N_GRAPHS, D), jnp.float32),
      ],
  )(px, a0, a1, Wa, ba.reshape(1, D), Wb, bb.reshape(1, D), batch3,
    Wl1, bl1.reshape(1, D), Wl2p, bl2b)
  return out


def kernel(x, edge_index, batch, W1a, b1a, W1b, b1b, W2a, b2a, W2b, b2b,
           Wl1, bl1, Wl2, bl2):
  n_edges = edge_index.shape[1]
  t_chunks = -(-n_edges // CHUNK)
  kpt = -(-t_chunks // (NW * IH)) * IH
  tp = NW * kpt
  ep = tp * CHUNK

  src = edge_index[0].astype(jnp.int32)
  dst = edge_index[1].astype(jnp.int32)
  pad = ep - n_edges
  ar = jnp.arange(pad, dtype=jnp.int32)
  src_p = jnp.concatenate([src, ar % N_NODES])
  dump = N_NODES + ar % (N_PAD - N_NODES)
  dst_p = jnp.concatenate([dst, dump])
  src2 = src_p.reshape(tp, CHUNK)
  dst2 = dst_p.reshape(tp, CHUNK)

  batch3 = batch.astype(jnp.int32).reshape(N_NODES // BLK, 1, BLK)
  Wl2p = jnp.pad(Wl2, ((0, 0), (0, D - Wl2.shape[1])))
  bl2b = jnp.broadcast_to(bl2.reshape(1, 1), (1, D))

  acc1 = _sc_aggregate(x, src2, dst2, kpt)
  h1 = _mlp(x, acc1[0], acc1[1], W1a, b1a, W1b, b1b)
  acc2 = _sc_aggregate(h1, src2, dst2, kpt)
  out = _mlp_pool_head(h1, acc2[0], acc2[1], W2a, b2a, W2b, b2b, batch3,
                       Wl1, bl1, Wl2p, bl2b)
  return out[:, :1]
